# Initial kernel scaffold; baseline (speedup 1.0000x reference)
import jax
import jax.numpy as jnp
from jax.experimental import pallas as pl


def kernel(x, sel_out):
    raise NotImplementedError("write your pallas kernel here")


# speedup vs baseline: 11.7699x; 11.7699x over previous
import jax
import jax.numpy as jnp
from jax.experimental import pallas as pl
from jax.experimental.pallas import tpu as pltpu

N_NEG = 64
BLK = 256


def _softplus(v):
    return jnp.maximum(v, 0.0) + jnp.log1p(jnp.exp(-jnp.abs(v)))


def _body(x_ref, selr_ref, selc_ref, out_ref):
    i = pl.program_id(0)
    blk, n = x_ref.shape
    x = x_ref[...]
    sel_r = selr_ref[...]
    sel_c = selc_ref[...]
    mask = sel_c != sel_r
    m = jnp.where(mask, x, 0.0)

    row_ids = jax.lax.broadcasted_iota(jnp.int32, (blk, n), 0) + i * blk
    col_ids = jax.lax.broadcasted_iota(jnp.int32, (blk, n), 1)
    diag = jnp.sum(jnp.where(row_ids == col_ids, x, 0.0), axis=1)
    pos_part = jnp.sum(_softplus(-diag))

    def step(t, lo):
        cand = lo + (1 << (30 - t))
        tau = jax.lax.bitcast_convert_type(cand, jnp.float32)
        cnt = jnp.sum((m > tau).astype(jnp.float32), axis=1, keepdims=True)
        return jnp.where(cnt >= N_NEG, cand, lo)

    lo = jnp.zeros((blk, 1), jnp.int32)
    lo = jax.lax.fori_loop(0, 31, step, lo)
    v64 = jax.lax.bitcast_convert_type(lo + 1, jnp.float32)

    cnt_strict = jnp.sum((m > v64).astype(jnp.float32), axis=1, keepdims=True)
    s = jnp.sum(jnp.where(m > v64, _softplus(m), 0.0), axis=1, keepdims=True)
    s = s + (N_NEG - cnt_strict) * _softplus(v64)
    neg_part = jnp.sum(s)

    contrib = pos_part / n + neg_part / (n * N_NEG)

    @pl.when(i == 0)
    def _():
        out_ref[0, 0] = 0.0

    out_ref[0, 0] += contrib


def kernel(x, sel_out):
    n = x.shape[0]
    blk = min(BLK, n)
    grid = n // blk
    out = pl.pallas_call(
        _body,
        grid=(grid,),
        in_specs=[
            pl.BlockSpec((blk, n), lambda i: (i, 0)),
            pl.BlockSpec((blk, 1), lambda i: (i, 0)),
            pl.BlockSpec((1, n), lambda i: (0, 0)),
        ],
        out_specs=pl.BlockSpec(memory_space=pltpu.SMEM),
        out_shape=jax.ShapeDtypeStruct((1, 1), jnp.float32),
    )(x, sel_out.reshape(n, 1), sel_out.reshape(1, n))
    return out.reshape(())
